# baseline (speedup 1.0000x reference)
import functools

import jax
import jax.numpy as jnp
from jax import lax
from jax.experimental import pallas as pl
from jax.experimental.pallas import tpu as pltpu
from jax.experimental.pallas import tpu_sc as plsc

_TOK = 16384
_J = 50
_D = 64
_NW = 32
_RPW = _TOK // _NW
_TCW = _RPW // 128
_NGRP = _TCW * _J
_G = 128

_mesh = plsc.VectorSubcoreMesh(core_axis_name="c", subcore_axis_name="s")

_BC = 16384
_NB = -(-1000000 // _BC)


@functools.partial(
    pl.pallas_call,
    out_shape=jax.ShapeDtypeStruct((500000, 128), jnp.float32),
    grid=(_NB,),
    in_specs=[pl.BlockSpec((_D, _BC), lambda i: (0, i))],
    out_specs=pl.BlockSpec((_BC // 2, 128), lambda i: (i, 0)),
)
def _tc_detile(x_ref, o_ref):
    z = x_ref[...].T.reshape(_BC // 2, 2, _D)
    o_ref[:, 0:_D] = z[:, 0, :]
    o_ref[:, _D:128] = z[:, 1, :]


@functools.partial(
    pl.kernel,
    out_type=jax.ShapeDtypeStruct((_J, 8, _TOK // 128, 8, 128), jnp.float32),
    mesh=_mesh,
    scratch_types=[
        pltpu.VMEM((_RPW * _J,), jnp.int32),
        *[pltpu.VMEM((_G,), jnp.int32) for _ in range(4)],
        *[pltpu.VMEM((_G, _D), jnp.float32) for _ in range(4)],
        *[pltpu.VMEM((8, 8, 129), jnp.float32) for _ in range(2)],
        *[pltpu.SemaphoreType.DMA for _ in range(6)],
    ],
    compiler_params=pltpu.CompilerParams(
        use_tc_tiling_on_sc=False, needs_layout_passes=False
    ),
)
def _embedding_gather(idx_hbm, table_hbm, out_hbm, idx_v,
                      idx_g0, idx_g1, idx_g2, idx_g3,
                      rows0, rows1, rows2, rows3, tb0, tb1,
                      gs0, gs1, gs2, gs3, os0, os1):
    idx_g = (idx_g0, idx_g1, idx_g2, idx_g3)
    rows = (rows0, rows1, rows2, rows3)
    tbuf = (tb0, tb1)
    gsem = (gs0, gs1, gs2, gs3)
    osem = (os0, os1)

    wid = lax.axis_index("s") * 2 + lax.axis_index("c")
    base = wid * (_RPW * _J)
    pltpu.sync_copy(idx_hbm.at[pl.ds(base, _RPW * _J)], idx_v)
    tc0 = wid * _TCW
    lanes = lax.iota(jnp.int32, 16)
    toksJ = [(c16 * 16 + lanes) * _J for c16 in range(8)]
    zeros = lanes - lanes
    dtr = [(k * 16 + lanes) >> 3 for k in range(4)]
    dr = [(k * 16 + lanes) & 7 for k in range(4)]

    def fire_gather(g, s):
        tcl = g // _J
        j = g - tcl * _J
        gbase = tcl * (128 * _J) + j
        for c16 in range(8):
            vals = plsc.load_gather(idx_v, [toksJ[c16] + gbase])
            idx_g[s][pl.ds(c16 * 16, 16)] = vals
        pltpu.async_copy(table_hbm.at[idx_g[s]], rows[s], gsem[s])

    def transpose(s, t):
        def tchunk(i, carry):
            c0 = i * 8
            for cc in range(8):
                c = c0 + cc
                cv = zeros + c
                for k in range(4):
                    v = rows[s][c, pl.ds(k * 16, 16)]
                    plsc.store_scatter(tbuf[t], [dtr[k], dr[k], cv], v)
            return carry

        lax.fori_loop(0, _G // 8, tchunk, 0)

    def wait_gather(s):
        pltpu.make_async_copy(table_hbm.at[idx_g[s]], rows[s], gsem[s]).wait()

    def fire_out(g, s):
        tcl = g // _J
        j = g - tcl * _J
        pltpu.async_copy(
            tbuf[s].at[:, :, pl.ds(0, 128)], out_hbm.at[j, :, tc0 + tcl],
            osem[s],
        )

    def wait_out(g, s):
        tcl = g // _J
        j = g - tcl * _J
        pltpu.make_async_copy(
            tbuf[s].at[:, :, pl.ds(0, 128)], out_hbm.at[j, :, tc0 + tcl],
            osem[s],
        ).wait()

    fire_gather(0, 0)
    fire_gather(1, 1)

    def quad(q, carry):
        for b in range(4):
            g = q * 4 + b
            t = b % 2

            @pl.when(g + 2 < _NGRP)
            def _():
                fire_gather(g + 2, (b + 2) % 4)

            wait_gather(b)

            @pl.when(g >= 2)
            def _():
                wait_out(g - 2, t)

            transpose(b, t)
            fire_out(g, t)
        return carry

    lax.fori_loop(0, _NGRP // 4, quad, 0)
    wait_out(_NGRP - 2, 0)
    wait_out(_NGRP - 1, 1)


def kernel(token_ids, weight):
    flat = token_ids.reshape(-1).astype(jnp.int32)
    w_lin = _tc_detile(weight.T).reshape(1000000, _D)
    out5 = _embedding_gather(flat, w_lin)
    return out5.transpose(2, 4, 0, 1, 3).reshape(_TOK, _J, _D)


# speedup vs baseline: 1.0135x; 1.0135x over previous
import functools

import jax
import jax.numpy as jnp
from jax import lax
from jax.experimental import pallas as pl
from jax.experimental.pallas import tpu as pltpu
from jax.experimental.pallas import tpu_sc as plsc

_TOK = 16384
_J = 50
_D = 64
_NW = 32
_RPW = _TOK // _NW
_TCW = _RPW // 128
_NGRP = _TCW * _J
_G = 128

_mesh = plsc.VectorSubcoreMesh(core_axis_name="c", subcore_axis_name="s")

_BC = 16384
_NB = -(-1000000 // _BC)


@functools.partial(
    pl.pallas_call,
    out_shape=jax.ShapeDtypeStruct((500000, 128), jnp.float32),
    grid=(_NB,),
    in_specs=[pl.BlockSpec((_D, _BC), lambda i: (0, i))],
    out_specs=pl.BlockSpec((_BC // 2, 128), lambda i: (i, 0)),
)
def _tc_detile(x_ref, o_ref):
    z = x_ref[...].T.reshape(_BC // 2, 2, _D)
    o_ref[:, 0:_D] = z[:, 0, :]
    o_ref[:, _D:128] = z[:, 1, :]


@functools.partial(
    pl.kernel,
    out_type=jax.ShapeDtypeStruct((_J, 8, _TOK // 128, 8, 128), jnp.float32),
    mesh=_mesh,
    scratch_types=[
        pltpu.VMEM((_RPW * _J,), jnp.int32),
        *[pltpu.VMEM((_G,), jnp.int32) for _ in range(2)],
        *[pltpu.VMEM((_G, _D), jnp.float32) for _ in range(2)],
        *[pltpu.VMEM((8, 8, 129), jnp.float32) for _ in range(2)],
        *[pltpu.SemaphoreType.DMA for _ in range(4)],
    ],
    compiler_params=pltpu.CompilerParams(
        use_tc_tiling_on_sc=False, needs_layout_passes=False
    ),
)
def _embedding_gather(idx_hbm, table_hbm, out_hbm, idx_v,
                      idx_g0, idx_g1, rows0, rows1, tb0, tb1,
                      gs0, gs1, os0, os1):
    idx_g = (idx_g0, idx_g1)
    rows = (rows0, rows1)
    tbuf = (tb0, tb1)
    gsem = (gs0, gs1)
    osem = (os0, os1)

    wid = lax.axis_index("s") * 2 + lax.axis_index("c")
    base = wid * (_RPW * _J)
    pltpu.sync_copy(idx_hbm.at[pl.ds(base, _RPW * _J)], idx_v)
    tc0 = wid * _TCW
    lanes = lax.iota(jnp.int32, 16)
    toksJ = [(c16 * 16 + lanes) * _J for c16 in range(8)]
    zeros = lanes - lanes
    dtr = [(k * 16 + lanes) >> 3 for k in range(4)]
    dr = [(k * 16 + lanes) & 7 for k in range(4)]

    def fire_gather(g, s):
        tcl = g // _J
        j = g - tcl * _J
        gbase = tcl * (128 * _J) + j
        for c16 in range(8):
            vals = plsc.load_gather(idx_v, [toksJ[c16] + gbase])
            idx_g[s][pl.ds(c16 * 16, 16)] = vals
        pltpu.async_copy(table_hbm.at[idx_g[s]], rows[s], gsem[s])

    def transpose(s, t):
        def tchunk(i, carry):
            c0 = i * 8
            for cc in range(8):
                c = c0 + cc
                cv = zeros + c
                for k in range(4):
                    v = rows[s][c, pl.ds(k * 16, 16)]
                    plsc.store_scatter(tbuf[t], [dtr[k], dr[k], cv], v)
            return carry

        lax.fori_loop(0, _G // 8, tchunk, 0)

    def wait_gather(s):
        pltpu.make_async_copy(table_hbm.at[idx_g[s]], rows[s], gsem[s]).wait()

    def fire_out(g, s):
        tcl = g // _J
        j = g - tcl * _J
        pltpu.async_copy(
            tbuf[s].at[:, :, pl.ds(0, 128)], out_hbm.at[j, :, tc0 + tcl],
            osem[s],
        )

    def wait_out(g, s):
        tcl = g // _J
        j = g - tcl * _J
        pltpu.make_async_copy(
            tbuf[s].at[:, :, pl.ds(0, 128)], out_hbm.at[j, :, tc0 + tcl],
            osem[s],
        ).wait()

    fire_gather(0, 0)

    def pair(gg, carry):
        for b in range(2):
            g = gg * 2 + b

            @pl.when(g + 1 < _NGRP)
            def _():
                fire_gather(g + 1, 1 - b)

            wait_gather(b)

            @pl.when(g >= 2)
            def _():
                wait_out(g - 2, b)

            transpose(b, b)
            fire_out(g, b)
        return carry

    lax.fori_loop(0, _NGRP // 2, pair, 0)
    wait_out(_NGRP - 2, 0)
    wait_out(_NGRP - 1, 1)


def kernel(token_ids, weight):
    flat = token_ids.reshape(-1).astype(jnp.int32)
    w_lin = _tc_detile(weight.T).reshape(1000000, _D)
    out5 = _embedding_gather(flat, w_lin)
    return out5.transpose(2, 4, 0, 1, 3).reshape(_TOK, _J, _D)
